# baseline (speedup 1.0000x reference)
import jax
import jax.numpy as jnp
from jax.experimental import pallas as pl
from jax.experimental.pallas import tpu as pltpu

_C = 91
_VALID = 80
_K = 100
_IOBLK = 2000
_SUB = 200
_PMAXN = 104


def _topk_kernel(logits_ref, obj_ref, boxes_ref,
                 scores_ref, labels_ref, boxes_out_ref,
                 fused_ref, rmax_ref, pmax_ref, parg_ref):
    nio = fused_ref.shape[0] // _IOBLK
    nsub_per_io = _IOBLK // _SUB
    j = pl.program_id(1)

    @pl.when(j < nio)
    def _phase1():
        lane_c = jax.lax.broadcasted_iota(jnp.int32, (_IOBLK, _C), 1)
        sub_s = jax.lax.broadcasted_iota(jnp.int32, (_SUB, 1), 0)
        lg = logits_ref[0]
        ob = obj_ref[0]
        f = ob * jax.nn.sigmoid(lg)
        f = jnp.where(lane_c < _VALID, f, 0.0)
        fused_ref[pl.ds(j * _IOBLK, _IOBLK), :] = f
        rm = jnp.max(f, axis=1, keepdims=True)
        rmax_ref[pl.ds(j * _IOBLK, _IOBLK), :] = rm
        for s in range(nsub_per_io):
            seg = rm[s * _SUB:(s + 1) * _SUB]
            sm = jnp.max(seg, axis=0, keepdims=True)
            ar = jnp.min(jnp.where(seg >= sm[0, 0], sub_s, _SUB),
                         axis=0, keepdims=True)
            base = j * _IOBLK + s * _SUB
            pmax_ref[pl.ds(j * nsub_per_io + s, 1), :] = sm
            parg_ref[pl.ds(j * nsub_per_io + s, 1), :] = ar + base

    @pl.when(j == nio)
    def _phase2():
        nsub = nio * nsub_per_io
        pad = _PMAXN - nsub
        pmax_ref[pl.ds(nsub, pad), :] = jnp.full((pad, 1), -2.0, jnp.float32)
        parg_ref[pl.ds(nsub, pad), :] = jnp.zeros((pad, 1), jnp.int32)

        sub_p = jax.lax.broadcasted_iota(jnp.int32, (_PMAXN, 1), 0)
        sub_s = jax.lax.broadcasted_iota(jnp.int32, (_SUB, 1), 0)
        lane128 = jax.lax.broadcasted_iota(jnp.int32, (1, 128), 1)
        lane_c1 = jax.lax.broadcasted_iota(jnp.int32, (1, _C), 1)
        qbig = fused_ref.shape[0]

        def round_body(i, carry):
            svec, lvec, pvec, avec = carry
            vmax2 = jnp.max(pvec, axis=0, keepdims=True)
            vmax = vmax2[0, 0]
            msk = pvec >= vmax
            q = jnp.min(jnp.where(msk, avec, qbig),
                        axis=0, keepdims=True)[0, 0]
            jsel = q // _SUB
            rsel = q - jsel * _SUB
            row = fused_ref[pl.ds(q, 1), :]
            csel = jnp.min(jnp.where(row >= vmax, lane_c1, _C),
                           axis=1, keepdims=True)[0, 0]

            svec = jnp.where(lane128 == i, vmax, svec)
            lvec = jnp.where(lane128 == i, csel, lvec)
            boxes_out_ref[0, pl.ds(i, 1), :] = boxes_ref[0, pl.ds(q, 1), :]

            newrow = jnp.where(lane_c1 == csel, -1.0, row)
            fused_ref[pl.ds(q, 1), :] = newrow
            nrmax = jnp.max(newrow, axis=1, keepdims=True)
            rmax_ref[pl.ds(q, 1), :] = nrmax

            rslab = rmax_ref[pl.ds(jsel * _SUB, _SUB), :]
            rslab = jnp.where(sub_s == rsel, nrmax[0, 0], rslab)
            sm2 = jnp.max(rslab, axis=0, keepdims=True)
            sm = sm2[0, 0]
            ar = jnp.min(jnp.where(rslab >= sm, sub_s, _SUB),
                         axis=0, keepdims=True)[0, 0]
            hit = sub_p == jsel
            pvec = jnp.where(hit, sm, pvec)
            avec = jnp.where(hit, jsel * _SUB + ar, avec)
            return svec, lvec, pvec, avec

        svec0 = jnp.zeros((1, 128), jnp.float32)
        lvec0 = jnp.zeros((1, 128), jnp.int32)
        pvec0 = pmax_ref[:, :]
        avec0 = parg_ref[:, :]
        svec, lvec, _, _ = jax.lax.fori_loop(
            0, _K, round_body, (svec0, lvec0, pvec0, avec0))
        scores_ref[0, :, :] = svec
        labels_ref[0, :, :] = lvec


def kernel(pred_logits, pred_obj, pred_boxes, target_sizes):
    B, Q, C = pred_logits.shape
    nio = Q // _IOBLK

    cx = pred_boxes[..., 0]
    cy = pred_boxes[..., 1]
    w = pred_boxes[..., 2]
    h = pred_boxes[..., 3]
    xyxy = jnp.stack([cx - 0.5 * w, cy - 0.5 * h,
                      cx + 0.5 * w, cy + 0.5 * h], axis=-1)
    ih = target_sizes[:, 0].astype(xyxy.dtype)
    iw = target_sizes[:, 1].astype(xyxy.dtype)
    scale = jnp.stack([iw, ih, iw, ih], axis=1)
    sboxes = xyxy * scale[:, None, :]
    objp = jnp.exp(-pred_obj)[..., None]

    scores, labels, boxes = pl.pallas_call(
        _topk_kernel,
        grid=(B, nio + 1),
        in_specs=[
            pl.BlockSpec((1, _IOBLK, C), lambda b, j: (b, jnp.minimum(j, nio - 1), 0)),
            pl.BlockSpec((1, _IOBLK, 1), lambda b, j: (b, jnp.minimum(j, nio - 1), 0)),
            pl.BlockSpec((1, Q, 4), lambda b, j: (b, 0, 0)),
        ],
        out_specs=[
            pl.BlockSpec((1, 1, 128), lambda b, j: (b, 0, 0)),
            pl.BlockSpec((1, 1, 128), lambda b, j: (b, 0, 0)),
            pl.BlockSpec((1, _K, 4), lambda b, j: (b, 0, 0)),
        ],
        out_shape=[
            jax.ShapeDtypeStruct((B, 1, 128), jnp.float32),
            jax.ShapeDtypeStruct((B, 1, 128), jnp.int32),
            jax.ShapeDtypeStruct((B, _K, 4), jnp.float32),
        ],
        scratch_shapes=[
            pltpu.VMEM((Q, _C), jnp.float32),
            pltpu.VMEM((Q, 1), jnp.float32),
            pltpu.VMEM((_PMAXN, 1), jnp.float32),
            pltpu.VMEM((_PMAXN, 1), jnp.int32),
        ],
        compiler_params=pltpu.CompilerParams(
            dimension_semantics=("parallel", "arbitrary"),
        ),
    )(pred_logits, objp, sboxes)

    return scores[:, 0, :_K], labels[:, 0, :_K], boxes


# speedup vs baseline: 1.0769x; 1.0769x over previous
import jax
import jax.numpy as jnp
from jax.experimental import pallas as pl
from jax.experimental.pallas import tpu as pltpu

_C = 91
_VALID = 80
_K = 100
_IOBLK = 4000
_SUB = 200
_PMAXN = 104


def _topk_kernel(logits_ref, obj_ref, boxes_ref,
                 scores_ref, labels_ref, boxes_out_ref,
                 fused_ref, rmax_ref, pmax_ref, parg_ref):
    nio = fused_ref.shape[0] // _IOBLK
    nsub_per_io = _IOBLK // _SUB
    j = pl.program_id(1)

    @pl.when(j < nio)
    def _phase1():
        lane_c = jax.lax.broadcasted_iota(jnp.int32, (_IOBLK, _C), 1)
        sub_s = jax.lax.broadcasted_iota(jnp.int32, (_SUB, 1), 0)
        lg = logits_ref[0]
        ob = obj_ref[0]
        f = ob * jax.nn.sigmoid(lg)
        f = jnp.where(lane_c < _VALID, f, 0.0)
        fused_ref[pl.ds(j * _IOBLK, _IOBLK), :] = f
        rm = jnp.max(f, axis=1, keepdims=True)
        rmax_ref[pl.ds(j * _IOBLK, _IOBLK), :] = rm
        for s in range(nsub_per_io):
            seg = rm[s * _SUB:(s + 1) * _SUB]
            sm = jnp.max(seg, axis=0, keepdims=True)
            ar = jnp.min(jnp.where(seg >= sm[0, 0], sub_s, _SUB),
                         axis=0, keepdims=True)
            base = j * _IOBLK + s * _SUB
            pmax_ref[pl.ds(j * nsub_per_io + s, 1), :] = sm
            parg_ref[pl.ds(j * nsub_per_io + s, 1), :] = ar + base

    @pl.when(j == nio)
    def _phase2():
        nsub = nio * nsub_per_io
        pad = _PMAXN - nsub
        pmax_ref[pl.ds(nsub, pad), :] = jnp.full((pad, 1), -2.0, jnp.float32)
        parg_ref[pl.ds(nsub, pad), :] = jnp.zeros((pad, 1), jnp.int32)

        sub_p = jax.lax.broadcasted_iota(jnp.int32, (_PMAXN, 1), 0)
        sub_s = jax.lax.broadcasted_iota(jnp.int32, (_SUB, 1), 0)
        lane128 = jax.lax.broadcasted_iota(jnp.int32, (1, 128), 1)
        lane_c1 = jax.lax.broadcasted_iota(jnp.int32, (1, _C), 1)
        qbig = fused_ref.shape[0]

        def round_body(i, carry):
            svec, lvec = carry
            p = pmax_ref[:, :]
            a = parg_ref[:, :]
            vmax2 = jnp.max(p, axis=0, keepdims=True)
            vmax = vmax2[0, 0]
            msk = p >= vmax
            q = jnp.min(jnp.where(msk, a, qbig),
                        axis=0, keepdims=True)[0, 0]
            jsel = q // _SUB
            rsel = q - jsel * _SUB
            row = fused_ref[pl.ds(q, 1), :]
            csel = jnp.min(jnp.where(row >= vmax, lane_c1, _C),
                           axis=1, keepdims=True)[0, 0]

            svec = jnp.where(lane128 == i, vmax, svec)
            lvec = jnp.where(lane128 == i, csel, lvec)
            boxes_out_ref[0, pl.ds(i, 1), :] = boxes_ref[0, pl.ds(q, 1), :]

            newrow = jnp.where(lane_c1 == csel, -1.0, row)
            fused_ref[pl.ds(q, 1), :] = newrow
            nrmax = jnp.max(newrow, axis=1, keepdims=True)
            rmax_ref[pl.ds(q, 1), :] = nrmax

            rslab = rmax_ref[pl.ds(jsel * _SUB, _SUB), :]
            rslab = jnp.where(sub_s == rsel, nrmax[0, 0], rslab)
            sm2 = jnp.max(rslab, axis=0, keepdims=True)
            sm = sm2[0, 0]
            ar = jnp.min(jnp.where(rslab >= sm, sub_s, _SUB),
                         axis=0, keepdims=True)
            pmax_ref[pl.ds(jsel, 1), :] = sm2
            parg_ref[pl.ds(jsel, 1), :] = ar + jsel * _SUB
            return svec, lvec

        svec0 = jnp.zeros((1, 128), jnp.float32)
        lvec0 = jnp.zeros((1, 128), jnp.int32)
        svec, lvec = jax.lax.fori_loop(0, _K, round_body, (svec0, lvec0))
        scores_ref[0, :, :] = svec
        labels_ref[0, :, :] = lvec


def kernel(pred_logits, pred_obj, pred_boxes, target_sizes):
    B, Q, C = pred_logits.shape
    nio = Q // _IOBLK

    cx = pred_boxes[..., 0]
    cy = pred_boxes[..., 1]
    w = pred_boxes[..., 2]
    h = pred_boxes[..., 3]
    xyxy = jnp.stack([cx - 0.5 * w, cy - 0.5 * h,
                      cx + 0.5 * w, cy + 0.5 * h], axis=-1)
    ih = target_sizes[:, 0].astype(xyxy.dtype)
    iw = target_sizes[:, 1].astype(xyxy.dtype)
    scale = jnp.stack([iw, ih, iw, ih], axis=1)
    sboxes = xyxy * scale[:, None, :]
    objp = jnp.exp(-pred_obj)[..., None]

    scores, labels, boxes = pl.pallas_call(
        _topk_kernel,
        grid=(B, nio + 1),
        in_specs=[
            pl.BlockSpec((1, _IOBLK, C), lambda b, j: (b, jnp.minimum(j, nio - 1), 0)),
            pl.BlockSpec((1, _IOBLK, 1), lambda b, j: (b, jnp.minimum(j, nio - 1), 0)),
            pl.BlockSpec((1, Q, 4), lambda b, j: (b, 0, 0)),
        ],
        out_specs=[
            pl.BlockSpec((1, 1, 128), lambda b, j: (b, 0, 0)),
            pl.BlockSpec((1, 1, 128), lambda b, j: (b, 0, 0)),
            pl.BlockSpec((1, _K, 4), lambda b, j: (b, 0, 0)),
        ],
        out_shape=[
            jax.ShapeDtypeStruct((B, 1, 128), jnp.float32),
            jax.ShapeDtypeStruct((B, 1, 128), jnp.int32),
            jax.ShapeDtypeStruct((B, _K, 4), jnp.float32),
        ],
        scratch_shapes=[
            pltpu.VMEM((Q, _C), jnp.float32),
            pltpu.VMEM((Q, 1), jnp.float32),
            pltpu.VMEM((_PMAXN, 1), jnp.float32),
            pltpu.VMEM((_PMAXN, 1), jnp.int32),
        ],
        compiler_params=pltpu.CompilerParams(
            dimension_semantics=("parallel", "arbitrary"),
        ),
    )(pred_logits, objp, sboxes)

    return scores[:, 0, :_K], labels[:, 0, :_K], boxes


# speedup vs baseline: 1.1035x; 1.0247x over previous
import jax
import jax.numpy as jnp
from jax.experimental import pallas as pl
from jax.experimental.pallas import tpu as pltpu

_C = 91
_VALID = 80
_K = 100
_IOBLK = 4000
_SUB = 200
_PMAXN = 104


def _topk_kernel(logits_ref, obj_ref, boxes_ref,
                 scores_ref, labels_ref, boxes_out_ref,
                 fused_ref, rmax_ref, pmax_ref, parg_ref):
    nio = fused_ref.shape[0] // _IOBLK
    nsub_per_io = _IOBLK // _SUB
    j = pl.program_id(1)

    @pl.when(j < nio)
    def _phase1():
        lane_c = jax.lax.broadcasted_iota(jnp.int32, (_IOBLK, _C), 1)
        sub_s = jax.lax.broadcasted_iota(jnp.int32, (_SUB, 1), 0)
        lg = logits_ref[0]
        ob = obj_ref[0]
        f = ob * jax.nn.sigmoid(lg)
        f = jnp.where(lane_c < _VALID, f, 0.0)
        fused_ref[pl.ds(j * _IOBLK, _IOBLK), :] = f
        rm = jnp.max(f, axis=1, keepdims=True)
        rmax_ref[pl.ds(j * _IOBLK, _IOBLK), :] = rm
        for s in range(nsub_per_io):
            seg = rm[s * _SUB:(s + 1) * _SUB]
            sm = jnp.max(seg, axis=0, keepdims=True)
            ar = jnp.min(jnp.where(seg >= sm[0, 0], sub_s, _SUB),
                         axis=0, keepdims=True)
            base = j * _IOBLK + s * _SUB
            pmax_ref[pl.ds(j * nsub_per_io + s, 1), :] = sm
            parg_ref[pl.ds(j * nsub_per_io + s, 1), :] = ar + base

    @pl.when(j == nio)
    def _phase2():
        nsub = nio * nsub_per_io
        pad = _PMAXN - nsub
        pmax_ref[pl.ds(nsub, pad), :] = jnp.full((pad, 1), -2.0, jnp.float32)
        parg_ref[pl.ds(nsub, pad), :] = jnp.zeros((pad, 1), jnp.int32)

        sub_p = jax.lax.broadcasted_iota(jnp.int32, (_PMAXN, 1), 0)
        sub_s = jax.lax.broadcasted_iota(jnp.int32, (_SUB, 1), 0)
        lane128 = jax.lax.broadcasted_iota(jnp.int32, (1, 128), 1)
        lane_c1 = jax.lax.broadcasted_iota(jnp.int32, (1, _C), 1)
        qbig = fused_ref.shape[0]

        def round_body(i, carry):
            svec, lvec = carry
            p = pmax_ref[:, :]
            a = parg_ref[:, :]
            vmax2 = jnp.max(p, axis=0, keepdims=True)
            vmax = vmax2[0, 0]
            msk = p >= vmax
            q = jnp.min(jnp.where(msk, a, qbig),
                        axis=0, keepdims=True)[0, 0]
            jsel = q // _SUB
            rsel = q - jsel * _SUB
            row = fused_ref[pl.ds(q, 1), :]
            csel = jnp.min(jnp.where(row >= vmax, lane_c1, _C),
                           axis=1, keepdims=True)[0, 0]

            svec = jnp.where(lane128 == i, vmax, svec)
            lvec = jnp.where(lane128 == i, csel, lvec)
            boxes_out_ref[0, pl.ds(i, 1), :] = boxes_ref[0, pl.ds(q, 1), :]

            newrow = jnp.where(lane_c1 == csel, -1.0, row)
            fused_ref[pl.ds(q, 1), :] = newrow
            nrmax = jnp.max(newrow, axis=1, keepdims=True)
            rmax_ref[pl.ds(q, 1), :] = nrmax

            rslab = rmax_ref[pl.ds(jsel * _SUB, _SUB), :]
            rslab = jnp.where(sub_s == rsel, nrmax[0, 0], rslab)
            sm2 = jnp.max(rslab, axis=0, keepdims=True)
            sm = sm2[0, 0]
            ar = jnp.min(jnp.where(rslab >= sm, sub_s, _SUB),
                         axis=0, keepdims=True)
            pmax_ref[pl.ds(jsel, 1), :] = sm2
            parg_ref[pl.ds(jsel, 1), :] = ar + jsel * _SUB
            return svec, lvec

        svec0 = jnp.zeros((1, 128), jnp.float32)
        lvec0 = jnp.zeros((1, 128), jnp.int32)
        svec, lvec = jax.lax.fori_loop(0, _K, round_body, (svec0, lvec0),
                                       unroll=2)
        scores_ref[0, :, :] = svec
        labels_ref[0, :, :] = lvec


def kernel(pred_logits, pred_obj, pred_boxes, target_sizes):
    B, Q, C = pred_logits.shape
    nio = Q // _IOBLK

    cx = pred_boxes[..., 0]
    cy = pred_boxes[..., 1]
    w = pred_boxes[..., 2]
    h = pred_boxes[..., 3]
    xyxy = jnp.stack([cx - 0.5 * w, cy - 0.5 * h,
                      cx + 0.5 * w, cy + 0.5 * h], axis=-1)
    ih = target_sizes[:, 0].astype(xyxy.dtype)
    iw = target_sizes[:, 1].astype(xyxy.dtype)
    scale = jnp.stack([iw, ih, iw, ih], axis=1)
    sboxes = xyxy * scale[:, None, :]
    objp = jnp.exp(-pred_obj)[..., None]

    scores, labels, boxes = pl.pallas_call(
        _topk_kernel,
        grid=(B, nio + 1),
        in_specs=[
            pl.BlockSpec((1, _IOBLK, C), lambda b, j: (b, jnp.minimum(j, nio - 1), 0)),
            pl.BlockSpec((1, _IOBLK, 1), lambda b, j: (b, jnp.minimum(j, nio - 1), 0)),
            pl.BlockSpec((1, Q, 4), lambda b, j: (b, 0, 0)),
        ],
        out_specs=[
            pl.BlockSpec((1, 1, 128), lambda b, j: (b, 0, 0)),
            pl.BlockSpec((1, 1, 128), lambda b, j: (b, 0, 0)),
            pl.BlockSpec((1, _K, 4), lambda b, j: (b, 0, 0)),
        ],
        out_shape=[
            jax.ShapeDtypeStruct((B, 1, 128), jnp.float32),
            jax.ShapeDtypeStruct((B, 1, 128), jnp.int32),
            jax.ShapeDtypeStruct((B, _K, 4), jnp.float32),
        ],
        scratch_shapes=[
            pltpu.VMEM((Q, _C), jnp.float32),
            pltpu.VMEM((Q, 1), jnp.float32),
            pltpu.VMEM((_PMAXN, 1), jnp.float32),
            pltpu.VMEM((_PMAXN, 1), jnp.int32),
        ],
        compiler_params=pltpu.CompilerParams(
            dimension_semantics=("parallel", "arbitrary"),
        ),
    )(pred_logits, objp, sboxes)

    return scores[:, 0, :_K], labels[:, 0, :_K], boxes


# speedup vs baseline: 1.1197x; 1.0146x over previous
import jax
import jax.numpy as jnp
from jax.experimental import pallas as pl
from jax.experimental.pallas import tpu as pltpu

_C = 91
_VALID = 80
_K = 100
_IOBLK = 4000
_SUB = 200
_PMAXN = 104


def _topk_kernel(logits_ref, obj_ref, boxes_ref,
                 scores_ref, labels_ref, boxes_out_ref,
                 fused_ref, rmax_ref, pmax_ref, parg_ref):
    nio = fused_ref.shape[0] // _IOBLK
    nsub_per_io = _IOBLK // _SUB
    j = pl.program_id(1)

    @pl.when(j < nio)
    def _phase1():
        lane_c = jax.lax.broadcasted_iota(jnp.int32, (_IOBLK, _C), 1)
        sub_s = jax.lax.broadcasted_iota(jnp.int32, (_SUB, 1), 0)
        lg = logits_ref[0]
        ob = obj_ref[0]
        f = ob * jax.nn.sigmoid(lg)
        f = jnp.where(lane_c < _VALID, f, 0.0)
        fused_ref[pl.ds(j * _IOBLK, _IOBLK), :] = f
        rm = jnp.max(f, axis=1, keepdims=True)
        rmax_ref[pl.ds(j * _IOBLK, _IOBLK), :] = rm
        for s in range(nsub_per_io):
            seg = rm[s * _SUB:(s + 1) * _SUB]
            sm = jnp.max(seg, axis=0, keepdims=True)
            ar = jnp.min(jnp.where(seg >= sm[0, 0], sub_s, _SUB),
                         axis=0, keepdims=True)
            base = j * _IOBLK + s * _SUB
            pmax_ref[pl.ds(j * nsub_per_io + s, 1), :] = sm
            parg_ref[pl.ds(j * nsub_per_io + s, 1), :] = ar + base

    @pl.when(j == nio)
    def _phase2():
        nsub = nio * nsub_per_io
        pad = _PMAXN - nsub
        pmax_ref[pl.ds(nsub, pad), :] = jnp.full((pad, 1), -2.0, jnp.float32)
        parg_ref[pl.ds(nsub, pad), :] = jnp.zeros((pad, 1), jnp.int32)

        sub_p = jax.lax.broadcasted_iota(jnp.int32, (_PMAXN, 1), 0)
        sub_s = jax.lax.broadcasted_iota(jnp.int32, (_SUB, 1), 0)
        lane128 = jax.lax.broadcasted_iota(jnp.int32, (1, 128), 1)
        lane_c1 = jax.lax.broadcasted_iota(jnp.int32, (1, _C), 1)
        qbig = fused_ref.shape[0]

        def round_body(i, carry):
            svec, lvec = carry
            p = pmax_ref[:, :]
            a = parg_ref[:, :]
            vmax2 = jnp.max(p, axis=0, keepdims=True)
            vmax = vmax2[0, 0]
            msk = p >= vmax
            q = jnp.min(jnp.where(msk, a, qbig),
                        axis=0, keepdims=True)[0, 0]
            jsel = q // _SUB
            rsel = q - jsel * _SUB
            row = fused_ref[pl.ds(q, 1), :]
            csel = jnp.min(jnp.where(row >= vmax, lane_c1, _C),
                           axis=1, keepdims=True)[0, 0]

            svec = jnp.where(lane128 == i, vmax, svec)
            lvec = jnp.where(lane128 == i, csel, lvec)
            boxes_out_ref[0, pl.ds(i, 1), :] = boxes_ref[0, pl.ds(q, 1), :]

            newrow = jnp.where(lane_c1 == csel, -1.0, row)
            fused_ref[pl.ds(q, 1), :] = newrow
            nrmax = jnp.max(newrow, axis=1, keepdims=True)
            rmax_ref[pl.ds(q, 1), :] = nrmax

            rslab = rmax_ref[pl.ds(jsel * _SUB, _SUB), :]
            rslab = jnp.where(sub_s == rsel, nrmax[0, 0], rslab)
            sm2 = jnp.max(rslab, axis=0, keepdims=True)
            sm = sm2[0, 0]
            ar = jnp.min(jnp.where(rslab >= sm, sub_s, _SUB),
                         axis=0, keepdims=True)
            pmax_ref[pl.ds(jsel, 1), :] = sm2
            parg_ref[pl.ds(jsel, 1), :] = ar + jsel * _SUB
            return svec, lvec

        svec0 = jnp.zeros((1, 128), jnp.float32)
        lvec0 = jnp.zeros((1, 128), jnp.int32)
        svec, lvec = jax.lax.fori_loop(0, _K, round_body, (svec0, lvec0),
                                       unroll=4)
        scores_ref[0, :, :] = svec
        labels_ref[0, :, :] = lvec


def kernel(pred_logits, pred_obj, pred_boxes, target_sizes):
    B, Q, C = pred_logits.shape
    nio = Q // _IOBLK

    cx = pred_boxes[..., 0]
    cy = pred_boxes[..., 1]
    w = pred_boxes[..., 2]
    h = pred_boxes[..., 3]
    xyxy = jnp.stack([cx - 0.5 * w, cy - 0.5 * h,
                      cx + 0.5 * w, cy + 0.5 * h], axis=-1)
    ih = target_sizes[:, 0].astype(xyxy.dtype)
    iw = target_sizes[:, 1].astype(xyxy.dtype)
    scale = jnp.stack([iw, ih, iw, ih], axis=1)
    sboxes = xyxy * scale[:, None, :]
    objp = jnp.exp(-pred_obj)[..., None]

    scores, labels, boxes = pl.pallas_call(
        _topk_kernel,
        grid=(B, nio + 1),
        in_specs=[
            pl.BlockSpec((1, _IOBLK, C), lambda b, j: (b, jnp.minimum(j, nio - 1), 0)),
            pl.BlockSpec((1, _IOBLK, 1), lambda b, j: (b, jnp.minimum(j, nio - 1), 0)),
            pl.BlockSpec((1, Q, 4), lambda b, j: (b, 0, 0)),
        ],
        out_specs=[
            pl.BlockSpec((1, 1, 128), lambda b, j: (b, 0, 0)),
            pl.BlockSpec((1, 1, 128), lambda b, j: (b, 0, 0)),
            pl.BlockSpec((1, _K, 4), lambda b, j: (b, 0, 0)),
        ],
        out_shape=[
            jax.ShapeDtypeStruct((B, 1, 128), jnp.float32),
            jax.ShapeDtypeStruct((B, 1, 128), jnp.int32),
            jax.ShapeDtypeStruct((B, _K, 4), jnp.float32),
        ],
        scratch_shapes=[
            pltpu.VMEM((Q, _C), jnp.float32),
            pltpu.VMEM((Q, 1), jnp.float32),
            pltpu.VMEM((_PMAXN, 1), jnp.float32),
            pltpu.VMEM((_PMAXN, 1), jnp.int32),
        ],
        compiler_params=pltpu.CompilerParams(
            dimension_semantics=("parallel", "arbitrary"),
        ),
    )(pred_logits, objp, sboxes)

    return scores[:, 0, :_K], labels[:, 0, :_K], boxes
